# baseline (speedup 1.0000x reference)
import jax
import jax.numpy as jnp
from jax import lax
from jax.experimental import pallas as pl
from jax.experimental.pallas import tpu as pltpu

_CH = 20000
_NBUF = 2


def _body(h_hbm, graph_ref, W_ref, b_ref, out_hbm, ibuf, obuf, isem, osem):
    n = h_hbm.shape[0]
    nch = n // _CH

    def in_copy(i, s):
        return pltpu.make_async_copy(
            h_hbm.at[pl.ds(i * _CH, _CH)], ibuf.at[s], isem.at[s])

    def out_copy(i, s):
        return pltpu.make_async_copy(
            obuf.at[s], out_hbm.at[pl.ds(i * _CH, _CH)], osem.at[s])

    for s in range(min(_NBUF, nch)):
        in_copy(s, s).start()

    M = lax.dot_general(
        W_ref[:, :], graph_ref[:, :], (((0,), (0,)), ((), ())),
        preferred_element_type=jnp.float32)
    bg = jnp.dot(
        b_ref[:, :], graph_ref[:, :], preferred_element_type=jnp.float32)

    for i in range(nch):
        s = i % _NBUF
        in_copy(i, s).wait()
        if i >= _NBUF:
            out_copy(i - _NBUF, s).wait()
        obuf[s] = jnp.dot(
            ibuf[s], M, preferred_element_type=jnp.float32) + bg
        out_copy(i, s).start()
        if i + _NBUF < nch:
            in_copy(i + _NBUF, s).start()

    for i in range(max(0, nch - _NBUF), nch):
        out_copy(i, i % _NBUF).wait()


def kernel(h, graph, W, b):
    Bb, T, D = h.shape
    G = graph.shape[1]
    n = Bb * T
    h2 = h.reshape(n, D)
    b2 = b.reshape(1, -1)
    out = pl.pallas_call(
        _body,
        in_specs=[
            pl.BlockSpec(memory_space=pl.ANY),
            pl.BlockSpec(memory_space=pltpu.VMEM),
            pl.BlockSpec(memory_space=pltpu.VMEM),
            pl.BlockSpec(memory_space=pltpu.VMEM),
        ],
        out_specs=pl.BlockSpec(memory_space=pl.ANY),
        out_shape=jax.ShapeDtypeStruct((n, G), jnp.float32),
        scratch_shapes=[
            pltpu.VMEM((_NBUF, _CH, D), jnp.float32),
            pltpu.VMEM((_NBUF, _CH, G), jnp.float32),
            pltpu.SemaphoreType.DMA((_NBUF,)),
            pltpu.SemaphoreType.DMA((_NBUF,)),
        ],
    )(h2, graph, W, b2)
    return out.reshape(Bb, T, G)


# speedup vs baseline: 1.0305x; 1.0305x over previous
import jax
import jax.numpy as jnp
from jax import lax
from jax.experimental import pallas as pl
from jax.experimental.pallas import tpu as pltpu

_BLOCK = 20000


def _body(h_ref, graph_ref, W_ref, b_ref, out_ref, M_ref, bg_ref):
    @pl.when(pl.program_id(0) == 0)
    def _():
        M_ref[:, :] = lax.dot_general(
            W_ref[:, :], graph_ref[:, :], (((0,), (0,)), ((), ())),
            preferred_element_type=jnp.float32)
        bg_ref[:, :] = jnp.dot(
            b_ref[:, :], graph_ref[:, :], preferred_element_type=jnp.float32)

    out_ref[:, :] = jnp.dot(
        h_ref[:, :], M_ref[:, :], preferred_element_type=jnp.float32,
        precision=lax.Precision.DEFAULT,
    ) + bg_ref[:, :]


def kernel(h, graph, W, b):
    Bb, T, D = h.shape
    G = graph.shape[1]
    n = Bb * T
    h2 = h.reshape(n, D)
    b2 = b.reshape(1, -1)
    out = pl.pallas_call(
        _body,
        grid=(n // _BLOCK,),
        in_specs=[
            pl.BlockSpec((_BLOCK, D), lambda i: (i, 0)),
            pl.BlockSpec(graph.shape, lambda i: (0, 0)),
            pl.BlockSpec(W.shape, lambda i: (0, 0)),
            pl.BlockSpec((1, G), lambda i: (0, 0)),
        ],
        out_specs=pl.BlockSpec((_BLOCK, G), lambda i: (i, 0)),
        out_shape=jax.ShapeDtypeStruct((n, G), jnp.float32),
        scratch_shapes=[
            pltpu.VMEM((W.shape[1], G), jnp.float32),
            pltpu.VMEM((1, G), jnp.float32),
        ],
        compiler_params=pltpu.CompilerParams(
            dimension_semantics=("arbitrary",)),
    )(h2, graph, W, b2)
    return out.reshape(Bb, T, G)
